# baseline (speedup 1.0000x reference)
import functools

import jax
import jax.numpy as jnp
from jax import lax
from jax.experimental import pallas as pl
from jax.experimental.pallas import tpu as pltpu
from jax.experimental.pallas import tpu_sc as plsc

_KPAD = 128


def _idx_body(p_ref, lrows_ref, hl2_ref, idx_ref):
    p = p_ref[...]
    lrows = lrows_ref[...]
    scoresT = hl2_ref[...] - jax.lax.dot_general(
        lrows, p, (((1,), (1,)), ((), ())),
        preferred_element_type=jnp.float32)
    m = jnp.min(scoresT, axis=0, keepdims=True)
    rows = jax.lax.broadcasted_iota(
        jnp.int32, scoresT.shape, 0).astype(jnp.float32)
    idx = jnp.min(jnp.where(scoresT == m, rows, float(_KPAD)), axis=0,
                  keepdims=True)
    idx_ref[...] = idx.astype(jnp.int32)


@functools.partial(jax.jit, static_argnames=("block",))
def _nearest_idx(params, lattice_points, block=16384):
    n, d = params.shape
    k = lattice_points.shape[0]
    lrows = jnp.zeros((_KPAD, d), jnp.float32).at[:k].set(lattice_points)
    hl2 = 0.5 * jnp.sum(lrows * lrows, axis=1)
    hl2 = jnp.where(jnp.arange(_KPAD) < k, hl2, jnp.inf)[:, None]
    grid = (n // block,)
    idx2d = pl.pallas_call(
        _idx_body,
        grid=grid,
        in_specs=[
            pl.BlockSpec((block, d), lambda i: (i, 0)),
            pl.BlockSpec((_KPAD, d), lambda i: (0, 0)),
            pl.BlockSpec((_KPAD, 1), lambda i: (0, 0)),
        ],
        out_specs=pl.BlockSpec((1, block), lambda i: (0, i)),
        out_shape=jax.ShapeDtypeStruct((1, n), jnp.int32),
    )(params, lrows, hl2)
    return idx2d.reshape(n)


_NC = 2
_NS = 16
_NW = _NC * _NS
_CHUNK = 512


_SUB = 128
_DPAD = 128


def _make_sc_gather(n, d):
    per_w = n // _NW
    nch = per_w // _CHUNK
    mesh = plsc.VectorSubcoreMesh(core_axis_name="c", subcore_axis_name="s")

    @functools.partial(
        pl.kernel, mesh=mesh,
        out_type=jax.ShapeDtypeStruct((n, _DPAD), jnp.float32),
        scratch_types=[
            pltpu.VMEM((per_w,), jnp.int32),
            pltpu.VMEM((_CHUNK, _DPAD), jnp.float32),
            pltpu.SemaphoreType.DMA,
        ],
    )
    def gather(table_hbm, idx_hbm, out_hbm, idx_v, rows_v, sem):
        wid = lax.axis_index("s") * _NC + lax.axis_index("c")
        base = wid * per_w
        pltpu.sync_copy(idx_hbm.at[pl.ds(base, per_w)], idx_v)

        def chunk(c, carry):
            off = c * _CHUNK
            copies = [
                pltpu.async_copy(
                    table_hbm.at[idx_v.at[pl.ds(off + j * _SUB, _SUB)]],
                    rows_v.at[pl.ds(j * _SUB, _SUB)],
                    sem)
                for j in range(_CHUNK // _SUB)
            ]
            for cp in copies:
                cp.wait()
            pltpu.sync_copy(rows_v, out_hbm.at[pl.ds(base + off, _CHUNK)])
            return carry

        lax.fori_loop(0, nch, chunk, 0)

    return gather


def kernel(params, lattice_points):
    n, d = params.shape
    k = lattice_points.shape[0]
    idx = _nearest_idx(params, lattice_points)
    table_pad = jnp.zeros((k, _DPAD), jnp.float32).at[:, :d].set(
        lattice_points)
    out_pad = _make_sc_gather(n, d)(table_pad, idx)
    return out_pad[:, :d]


# speedup vs baseline: 3.6696x; 3.6696x over previous
import functools

import jax
import jax.numpy as jnp
from jax import lax
from jax.experimental import pallas as pl
from jax.experimental.pallas import tpu as pltpu
from jax.experimental.pallas import tpu_sc as plsc

_KPAD = 104


def _idx_body(p_ref, lrows_ref, hl2_ref, idx_ref):
    p = p_ref[...]
    lrows = lrows_ref[...]
    scoresT = hl2_ref[...] - jax.lax.dot_general(
        lrows, p, (((1,), (1,)), ((), ())),
        preferred_element_type=jnp.float32)
    m = jnp.min(scoresT, axis=0, keepdims=True)
    rows = jax.lax.broadcasted_iota(
        jnp.int32, scoresT.shape, 0).astype(jnp.float32)
    idx = jnp.min(jnp.where(scoresT == m, rows, float(_KPAD)), axis=0,
                  keepdims=True)
    idx_ref[...] = idx.astype(jnp.int32)


@functools.partial(jax.jit, static_argnames=("block",))
def _nearest_idx(params, lattice_points, block=16384):
    n, d = params.shape
    k = lattice_points.shape[0]
    lrows = jnp.zeros((_KPAD, d), jnp.float32).at[:k].set(lattice_points)
    hl2 = 0.5 * jnp.sum(lrows * lrows, axis=1)
    hl2 = jnp.where(jnp.arange(_KPAD) < k, hl2, jnp.inf)[:, None]
    grid = (n // block,)
    idx2d = pl.pallas_call(
        _idx_body,
        grid=grid,
        in_specs=[
            pl.BlockSpec((block, d), lambda i: (i, 0)),
            pl.BlockSpec((_KPAD, d), lambda i: (0, 0)),
            pl.BlockSpec((_KPAD, 1), lambda i: (0, 0)),
        ],
        out_specs=pl.BlockSpec((1, block), lambda i: (0, i)),
        out_shape=jax.ShapeDtypeStruct((1, n), jnp.int32),
    )(params, lrows, hl2)
    return idx2d.reshape(n)


_NC = 2
_NS = 16
_NW = _NC * _NS
_CH = 2048


def _make_sc_gather(n, d, k):
    per_w = n // _NW
    nch = per_w // _CH
    tflat = k * d
    mesh = plsc.VectorSubcoreMesh(core_axis_name="c", subcore_axis_name="s")

    @functools.partial(
        pl.kernel, mesh=mesh,
        out_type=jax.ShapeDtypeStruct((n * d,), jnp.float32),
        scratch_types=[
            pltpu.VMEM((tflat,), jnp.float32),
            pltpu.VMEM((per_w,), jnp.int32),
            pltpu.VMEM((_CH * d,), jnp.float32),
            pltpu.VMEM((_CH * d,), jnp.float32),
            pltpu.SemaphoreType.DMA,
        ],
        compiler_params=pltpu.CompilerParams(needs_layout_passes=False),
    )
    def gather(table_hbm, idx_hbm, out_hbm, table_v, idx_v, rows0, rows1,
               sem):
        wid = lax.axis_index("s") * _NC + lax.axis_index("c")
        base = wid * per_w
        pltpu.sync_copy(table_hbm, table_v)
        pltpu.sync_copy(idx_hbm.at[pl.ds(base, per_w)], idx_v)
        lane = lax.iota(jnp.int32, 16)

        def fill(rows_v, off):
            @plsc.parallel_loop(0, _CH // 16, unroll=4)
            def group(j):
                iv = idx_v[pl.ds(off + j * 16, 16)]
                word = iv * d
                dst = (j * 16 + lane) * d
                for dd in range(d):
                    v = plsc.load_gather(table_v, [word + dd])
                    plsc.store_scatter(rows_v, [dst + dd], v)

        bufs = (rows0, rows1)
        pending = [None, None]
        for c in range(nch):
            rv = bufs[c % 2]
            if pending[c % 2] is not None:
                pending[c % 2].wait()
            fill(rv, c * _CH)
            pending[c % 2] = pltpu.async_copy(
                rv, out_hbm.at[pl.ds((base + c * _CH) * d, _CH * d)], sem)
        pending[0].wait()
        pending[1].wait()

    return gather


def kernel(params, lattice_points):
    n, d = params.shape
    k = lattice_points.shape[0]
    idx = _nearest_idx(params, lattice_points)
    table_flat = lattice_points.reshape(k * d)
    flat = _make_sc_gather(n, d, k)(table_flat, idx)
    return flat.reshape(n, d)
